# Initial kernel scaffold; baseline (speedup 1.0000x reference)
import jax
import jax.numpy as jnp
from jax.experimental import pallas as pl


def kernel(x, edge_index, W, a):
    raise NotImplementedError("write your pallas kernel here")


# speedup vs baseline: 7.1761x; 7.1761x over previous
import functools

import jax
import jax.numpy as jnp
from jax import lax
from jax.experimental import pallas as pl
from jax.experimental.pallas import tpu as pltpu
from jax.experimental.pallas import tpu_sc as plsc

_N = 10000
_E = 320000
_D = 128

_NC = 2
_NS = 16
_L = 16
_NW = _NC * _NS
_K = 128
_CH = -(-_E // (_NW * _K))
_EP = _NW * _CH * _K
_NPAD = 10112
_ROWS = _NPAD // _NS

_BA = 1024
_BC = 1024


def _attn_body(x_ref, w_ref, a_ref, h_ref, s1_ref, s2_ref, cself_ref):
    hb = jnp.dot(x_ref[...], w_ref[...], preferred_element_type=jnp.float32)
    h_ref[...] = hb
    av = a_ref[0, :]
    s1 = jnp.dot(hb, av[:_D])
    s2 = jnp.dot(hb, av[_D:])
    e = s1 + s2
    s1_ref[...] = s1
    s2_ref[...] = s2
    cself_ref[...] = jnp.exp(jnp.maximum(e, 0.2 * e))


def _attn_call(x, W, a):
    grid = (-(-_N // _BA),)
    vec_spec = pl.BlockSpec((_BA,), lambda i: (i,))
    vec_shape = jax.ShapeDtypeStruct((_N,), jnp.float32)
    return pl.pallas_call(
        _attn_body,
        grid=grid,
        in_specs=[
            pl.BlockSpec((_BA, _D), lambda i: (i, 0)),
            pl.BlockSpec((_D, _D), lambda i: (0, 0)),
            pl.BlockSpec((1, 2 * _D), lambda i: (0, 0)),
        ],
        out_specs=[
            pl.BlockSpec((_BA, _D), lambda i: (i, 0)),
            vec_spec, vec_spec, vec_spec,
        ],
        out_shape=[
            jax.ShapeDtypeStruct((_N, _D), jnp.float32),
            vec_shape, vec_shape, vec_shape,
        ],
    )(x, W, a)


def _edge_body(h_hbm, s1_hbm, s2_hbm, src_hbm, dst_hbm, z128_hbm, z1_hbm,
               num_out, den_out,
               s1_v, s2_v, src_c, dst_c, c_c, rows_v, num_sh, den_sh, sem):
    cid = lax.axis_index("c")
    sid = lax.axis_index("s")
    wid = cid * _NS + sid
    r0 = sid * _ROWS

    pltpu.sync_copy(z128_hbm.at[pl.ds(r0, _ROWS)], num_sh.at[pl.ds(r0, _ROWS)])
    pltpu.sync_copy(z1_hbm.at[pl.ds(0, _ROWS)], s1_v.at[pl.ds(0, _ROWS)])
    pltpu.sync_copy(s1_v.at[pl.ds(0, _ROWS)], den_sh.at[pl.ds(r0, _ROWS)])

    pltpu.sync_copy(s1_hbm, s1_v.at[pl.ds(0, _N)])
    pltpu.sync_copy(s2_hbm, s2_v.at[pl.ds(0, _N)])
    plsc.subcore_barrier()

    def chunk_body(j, carry):
        pltpu.sync_copy(src_hbm.at[wid, j], src_c)
        pltpu.sync_copy(dst_hbm.at[wid, j], dst_c)
        gather_copy = pltpu.async_copy(h_hbm.at[dst_c], rows_v, sem)
        for i in range(_K // _L):
            sv = src_c[pl.ds(i * _L, _L)]
            dv = dst_c[pl.ds(i * _L, _L)]
            s1g = plsc.load_gather(s1_v, [sv])
            s2g = plsc.load_gather(s2_v, [dv])
            e = s1g + s2g
            e = jnp.maximum(e, 0.2 * e)
            c = jnp.where(sv != dv, jnp.exp(e), 0.0)
            c_c[pl.ds(i * _L, _L)] = c
        gather_copy.wait()

        def scale_group(g, c2):
            cg = c_c[pl.ds(g * _L, _L)]
            for r16 in range(_L):
                r = g * _L + r16
                cb = jnp.broadcast_to(cg[r16], (_L,))
                for q in range(_D // _L):
                    rows_v[r, pl.ds(q * _L, _L)] = rows_v[r, pl.ds(q * _L, _L)] * cb
            return c2
        lax.fori_loop(0, _K // _L, scale_group, 0)

        pltpu.sync_copy(rows_v, num_sh.at[src_c], add=True)
        pltpu.sync_copy(c_c, den_sh.at[src_c], add=True)
        return carry
    lax.fori_loop(0, _CH, chunk_body, 0)

    plsc.subcore_barrier()

    pltpu.sync_copy(num_sh.at[pl.ds(r0, _ROWS)], num_out.at[cid, pl.ds(r0, _ROWS)])
    pltpu.sync_copy(den_sh.at[pl.ds(r0, _ROWS)], s1_v.at[pl.ds(0, _ROWS)])
    pltpu.sync_copy(s1_v.at[pl.ds(0, _ROWS)],
                    den_out.at[pl.ds(cid * _NPAD + r0, _ROWS)])


def _edge_call(h, s1, s2, srcp, dstp, z128, z1):
    mesh = plsc.VectorSubcoreMesh(
        core_axis_name="c", subcore_axis_name="s",
        num_cores=_NC, num_subcores=_NS)
    return pl.kernel(
        _edge_body,
        out_type=(
            jax.ShapeDtypeStruct((_NC, _NPAD, _D), jnp.float32),
            jax.ShapeDtypeStruct((_NC * _NPAD,), jnp.float32),
        ),
        mesh=mesh,
        scratch_types=[
            pltpu.VMEM((_NPAD,), jnp.float32),
            pltpu.VMEM((_NPAD,), jnp.float32),
            pltpu.VMEM((_K,), jnp.int32),
            pltpu.VMEM((_K,), jnp.int32),
            pltpu.VMEM((_K,), jnp.float32),
            pltpu.VMEM((_K, _D), jnp.float32),
            pltpu.VMEM_SHARED((_NPAD, _D), jnp.float32),
            pltpu.VMEM_SHARED((_NPAD,), jnp.float32),
            pltpu.SemaphoreType.DMA,
        ],
        compiler_params=pltpu.CompilerParams(needs_layout_passes=False),
    )(h, s1, s2, srcp, dstp, z128, z1)


def _combine_body(num_ref, den_ref, h_ref, cself_ref, out_ref):
    cself = cself_ref[...]
    numsum = num_ref[0] + num_ref[1] + cself[:, None] * h_ref[...]
    densum = den_ref[0] + den_ref[1] + cself
    out_ref[...] = numsum / densum[:, None]


def _combine_call(num, den, h, cself):
    grid = (-(-_N // _BC),)
    return pl.pallas_call(
        _combine_body,
        grid=grid,
        in_specs=[
            pl.BlockSpec((_NC, _BC, _D), lambda i: (0, i, 0)),
            pl.BlockSpec((_NC, _BC), lambda i: (0, i)),
            pl.BlockSpec((_BC, _D), lambda i: (i, 0)),
            pl.BlockSpec((_BC,), lambda i: (i,)),
        ],
        out_specs=pl.BlockSpec((_BC, _D), lambda i: (i, 0)),
        out_shape=jax.ShapeDtypeStruct((_N, _D), jnp.float32),
    )(num, den, h, cself)


def kernel(x, edge_index, W, a):
    src = edge_index[0].astype(jnp.int32)
    dst = edge_index[1].astype(jnp.int32)
    pad = _EP - _E
    srcp = jnp.concatenate([src, jnp.zeros((pad,), jnp.int32)]).reshape(_NW, _CH, _K)
    dstp = jnp.concatenate([dst, jnp.zeros((pad,), jnp.int32)]).reshape(_NW, _CH, _K)

    h, s1, s2, cself = _attn_call(x, W, a)

    z128 = jnp.zeros((_NPAD, _D), jnp.float32)
    z1 = jnp.zeros((_NPAD,), jnp.float32)
    num, den = _edge_call(h, s1, s2, srcp, dstp, z128, z1)
    den = den.reshape(_NC, _NPAD)

    return _combine_call(num[:, :_N], den[:, :_N], h, cself)


# speedup vs baseline: 10.9384x; 1.5243x over previous
import functools

import jax
import jax.numpy as jnp
from jax import lax
from jax.experimental import pallas as pl
from jax.experimental.pallas import tpu as pltpu
from jax.experimental.pallas import tpu_sc as plsc

_N = 10000
_E = 320000
_D = 128

_NC = 2
_NS = 16
_L = 16
_NW = _NC * _NS
_K = 96
_CH = -(-_E // (_NW * _K))
_EP = _NW * _CH * _K
_NSLOT = 3
_NPAD = 10112
_ROWS = _NPAD // _NS

_BA = 1024
_BC = 1024


def _attn_body(x_ref, w_ref, a_ref, h_ref, s1_ref, s2_ref, cself_ref):
    hb = jnp.dot(x_ref[...], w_ref[...], preferred_element_type=jnp.float32)
    h_ref[...] = hb
    av = a_ref[0, :]
    s1 = jnp.dot(hb, av[:_D])
    s2 = jnp.dot(hb, av[_D:])
    e = s1 + s2
    s1_ref[...] = s1
    s2_ref[...] = s2
    cself_ref[...] = jnp.exp(jnp.maximum(e, 0.2 * e))


def _attn_call(x, W, a):
    grid = (-(-_N // _BA),)
    vec_spec = pl.BlockSpec((_BA,), lambda i: (i,))
    vec_shape = jax.ShapeDtypeStruct((_N,), jnp.float32)
    return pl.pallas_call(
        _attn_body,
        grid=grid,
        in_specs=[
            pl.BlockSpec((_BA, _D), lambda i: (i, 0)),
            pl.BlockSpec((_D, _D), lambda i: (0, 0)),
            pl.BlockSpec((1, 2 * _D), lambda i: (0, 0)),
        ],
        out_specs=[
            pl.BlockSpec((_BA, _D), lambda i: (i, 0)),
            vec_spec, vec_spec, vec_spec,
        ],
        out_shape=[
            jax.ShapeDtypeStruct((_N, _D), jnp.float32),
            vec_shape, vec_shape, vec_shape,
        ],
    )(x, W, a)


def _edge_body(h_hbm, s1_hbm, s2_hbm, src_hbm, dst_hbm, z128_hbm, z1_hbm,
               num_out, den_out,
               src_c, dst_c, c_c, s1g_v, s2g_v, rows_v, bounce_v,
               num_sh, den_sh, rsem, ssem, wsem):
    cid = lax.axis_index("c")
    sid = lax.axis_index("s")
    wid = cid * _NS + sid
    r0 = sid * _ROWS

    pltpu.sync_copy(z128_hbm.at[pl.ds(r0, _ROWS)], num_sh.at[pl.ds(r0, _ROWS)])
    pltpu.sync_copy(z1_hbm.at[pl.ds(0, _ROWS)], bounce_v)
    pltpu.sync_copy(bounce_v, den_sh.at[pl.ds(r0, _ROWS)])
    plsc.subcore_barrier()


    def launch(j, k):
        pltpu.sync_copy(src_hbm.at[wid, j], src_c.at[k])
        pltpu.sync_copy(dst_hbm.at[wid, j], dst_c.at[k])
        pltpu.async_copy(h_hbm.at[dst_c.at[k]], rows_v.at[k], rsem.at[k])
        pltpu.async_copy(s1_hbm.at[src_c.at[k]], s1g_v.at[k], ssem.at[k])
        pltpu.async_copy(s2_hbm.at[dst_c.at[k]], s2g_v.at[k], ssem.at[k])

    def wait_gathers(k):
        pltpu.make_async_copy(s1_hbm.at[src_c.at[k]], s1g_v.at[k], ssem.at[k]).wait()
        pltpu.make_async_copy(s2_hbm.at[dst_c.at[k]], s2g_v.at[k], ssem.at[k]).wait()

    def wait_rows(k):
        pltpu.make_async_copy(h_hbm.at[dst_c.at[k]], rows_v.at[k], rsem.at[k]).wait()

    def start_scatter(k):
        pltpu.async_copy(rows_v.at[k], num_sh.at[src_c.at[k]], wsem.at[k], add=True)
        pltpu.async_copy(c_c.at[k], den_sh.at[src_c.at[k]], wsem.at[k], add=True)

    def wait_scatter(k):
        pltpu.make_async_copy(rows_v.at[k], num_sh.at[src_c.at[k]], wsem.at[k]).wait()
        pltpu.make_async_copy(c_c.at[k], den_sh.at[src_c.at[k]], wsem.at[k]).wait()

    def compute_c(k):
        for i in range(_K // _L):
            sv = src_c[k, pl.ds(i * _L, _L)]
            dv = dst_c[k, pl.ds(i * _L, _L)]
            e = s1g_v[k, pl.ds(i * _L, _L)] + s2g_v[k, pl.ds(i * _L, _L)]
            e = jnp.maximum(e, 0.2 * e)
            c = jnp.where(sv != dv, jnp.exp(e), 0.0)
            c_c[k, pl.ds(i * _L, _L)] = c

    def scale(k):
        def scale_group(g, c2):
            cg = c_c[k, pl.ds(g * _L, _L)]
            for r16 in range(_L):
                r = g * _L + r16
                cb = jnp.broadcast_to(cg[r16], (_L,))
                for q in range(_D // _L):
                    rows_v[k, r, pl.ds(q * _L, _L)] = (
                        rows_v[k, r, pl.ds(q * _L, _L)] * cb)
            return c2
        lax.fori_loop(0, _K // _L, scale_group, 0)

    def sub(j, t, kk, cur, nxt):
        if kk == 2:
            wait_scatter(nxt)
        else:
            @pl.when(t > 0)
            def _():
                wait_scatter(nxt)
        if kk == 2:
            @pl.when(t < _CH // _NSLOT - 1)
            def _():
                launch(j + 1, nxt)
        else:
            launch(j + 1, nxt)
        wait_gathers(cur)
        compute_c(cur)
        wait_rows(cur)
        scale(cur)
        start_scatter(cur)

    launch(0, 0)

    def body(t, carry):
        sub(3 * t + 0, t, 0, 0, 1)
        sub(3 * t + 1, t, 1, 1, 2)
        sub(3 * t + 2, t, 2, 2, 0)
        return carry
    lax.fori_loop(0, _CH // _NSLOT, body, 0)

    wait_scatter((_CH - 2) % _NSLOT)
    wait_scatter((_CH - 1) % _NSLOT)

    plsc.subcore_barrier()

    pltpu.sync_copy(num_sh.at[pl.ds(r0, _ROWS)], num_out.at[cid, pl.ds(r0, _ROWS)])
    pltpu.sync_copy(den_sh.at[pl.ds(r0, _ROWS)], bounce_v)
    pltpu.sync_copy(bounce_v,
                    den_out.at[pl.ds(cid * _NPAD + r0, _ROWS)])


def _edge_call(h, s1, s2, srcp, dstp, z128, z1):
    mesh = plsc.VectorSubcoreMesh(
        core_axis_name="c", subcore_axis_name="s",
        num_cores=_NC, num_subcores=_NS)
    return pl.kernel(
        _edge_body,
        out_type=(
            jax.ShapeDtypeStruct((_NC, _NPAD, _D), jnp.float32),
            jax.ShapeDtypeStruct((_NC * _NPAD,), jnp.float32),
        ),
        mesh=mesh,
        scratch_types=[
            pltpu.VMEM((_NSLOT, _K), jnp.int32),
            pltpu.VMEM((_NSLOT, _K), jnp.int32),
            pltpu.VMEM((_NSLOT, _K), jnp.float32),
            pltpu.VMEM((_NSLOT, _K), jnp.float32),
            pltpu.VMEM((_NSLOT, _K), jnp.float32),
            pltpu.VMEM((_NSLOT, _K, _D), jnp.float32),
            pltpu.VMEM((_ROWS,), jnp.float32),
            pltpu.VMEM_SHARED((_NPAD, _D), jnp.float32),
            pltpu.VMEM_SHARED((_NPAD,), jnp.float32),
            pltpu.SemaphoreType.DMA((_NSLOT,)),
            pltpu.SemaphoreType.DMA((_NSLOT,)),
            pltpu.SemaphoreType.DMA((_NSLOT,)),
        ],
        compiler_params=pltpu.CompilerParams(needs_layout_passes=False),
    )(h, s1, s2, srcp, dstp, z128, z1)


def _combine_body(num_ref, den_ref, h_ref, cself_ref, out_ref):
    cself = cself_ref[...]
    numsum = num_ref[0] + num_ref[1] + cself[:, None] * h_ref[...]
    densum = den_ref[0] + den_ref[1] + cself
    out_ref[...] = numsum / densum[:, None]


def _combine_call(num, den, h, cself):
    grid = (-(-_N // _BC),)
    return pl.pallas_call(
        _combine_body,
        grid=grid,
        in_specs=[
            pl.BlockSpec((_NC, _BC, _D), lambda i: (0, i, 0)),
            pl.BlockSpec((_NC, _BC), lambda i: (0, i)),
            pl.BlockSpec((_BC, _D), lambda i: (i, 0)),
            pl.BlockSpec((_BC,), lambda i: (i,)),
        ],
        out_specs=pl.BlockSpec((_BC, _D), lambda i: (i, 0)),
        out_shape=jax.ShapeDtypeStruct((_N, _D), jnp.float32),
    )(num, den, h, cself)


def kernel(x, edge_index, W, a):
    src = edge_index[0].astype(jnp.int32)
    dst = edge_index[1].astype(jnp.int32)
    pad = _EP - _E
    srcp = jnp.concatenate([src, jnp.zeros((pad,), jnp.int32)]).reshape(_NW, _CH, _K)
    dstp = jnp.concatenate([dst, jnp.zeros((pad,), jnp.int32)]).reshape(_NW, _CH, _K)

    h, s1, s2, cself = _attn_call(x, W, a)

    z128 = jnp.zeros((_NPAD, _D), jnp.float32)
    z1 = jnp.zeros((_NPAD,), jnp.float32)
    num, den = _edge_call(h, s1, s2, srcp, dstp, z128, z1)
    den = den.reshape(_NC, _NPAD)

    return _combine_call(num[:, :_N], den[:, :_N], h, cself)
